# baseline (speedup 1.0000x reference)
import functools

import jax
import jax.numpy as jnp
from jax import lax
from jax.experimental import pallas as pl
from jax.experimental.pallas import tpu as pltpu
from jax.experimental.pallas import tpu_sc as plsc

N = 100000
E = 1600000
NC = 2
NS = 16
NT = NC * NS
SL = 6272
NP = NS * SL
EB = 2560
NEB = E // EB
VPB = EB // 16


def _deg_body(dst_hbm, ew_hbm, out_hbm, deg_v, dst_b, ew_b):
    z16 = jnp.zeros((16,), jnp.float32)
    cid = lax.axis_index("c")
    sid = lax.axis_index("s")
    wid = sid * NC + cid

    def zero_deg(j, carry):
        deg_v[pl.ds(j * 16, 16)] = z16
        return carry

    lax.fori_loop(0, NP // 16, zero_deg, 0)

    nblk = jnp.where(wid < NEB % NT, NEB // NT + 1, NEB // NT)

    def block(b, carry):
        start = (wid + b * NT) * EB
        pltpu.sync_copy(dst_hbm.at[pl.ds(start, EB)], dst_b)
        pltpu.sync_copy(ew_hbm.at[pl.ds(start, EB)], ew_b)

        def vec(v, c2):
            d = dst_b[pl.ds(v * 16, 16)]
            w = ew_b[pl.ds(v * 16, 16)]
            plsc.addupdate_scatter(deg_v, [d], w)
            return c2

        return lax.fori_loop(0, VPB, vec, carry)

    lax.fori_loop(0, nblk, block, 0)
    pltpu.sync_copy(deg_v, out_hbm.at[pl.ds(wid * NP, NP)])


_deg_call = pl.kernel(
    _deg_body,
    out_type=jax.ShapeDtypeStruct((NT * NP,), jnp.float32),
    mesh=plsc.VectorSubcoreMesh(
        core_axis_name="c", subcore_axis_name="s", num_cores=NC,
        num_subcores=NS),
    scratch_types=[
        pltpu.VMEM((NP,), jnp.float32),
        pltpu.VMEM((EB,), jnp.int32),
        pltpu.VMEM((EB,), jnp.float32),
    ],
    compiler_params=pltpu.CompilerParams(needs_layout_passes=False),
)


G = 128
CB = EB + G


def _spmm_body(D, C, SP, src_hbm, dst_hbm, nrm_hbm, h_hbm, s_hbm,
               src_b, dst_b, nrm_b, csrc, cloc, cnrm, idx2d, rows_v, rows_w,
               chunk_sh, sem, sem2, semd, sems, semn):
    z16f = jnp.zeros((16,), jnp.float32)
    z16i = jnp.zeros((16,), jnp.int32)
    cid = lax.axis_index("c")
    sid = lax.axis_index("s")
    nchunks = SP // C
    rpt = C // NS
    my_chunks = (nchunks - cid + NC - 1) // NC
    nblk = jnp.where(sid < NEB % NS, NEB // NS + 1, NEB // NS)

    def chunk_iter(ci, carry0):
        lo = (ci * NC + cid) * C

        def zrow(r, cc):
            for j in range(D // 16):
                rows_v[r, pl.ds(j * 16, 16)] = z16f
            return cc

        lax.fori_loop(0, G, zrow, 0)

        def zchunk(k, cc):
            pltpu.sync_copy(rows_v.at[pl.ds(0, 64)],
                            chunk_sh.at[pl.ds(sid * rpt + k * 64, 64)])
            return cc

        lax.fori_loop(0, rpt // 64, zchunk, 0)
        plsc.subcore_barrier()

        def block(b, cc):
            start = (sid + b * NS) * EB
            dd = pltpu.async_copy(dst_hbm.at[pl.ds(start, EB)], dst_b, semd)
            ds_ = pltpu.async_copy(src_hbm.at[pl.ds(start, EB)], src_b, sems)
            dn = pltpu.async_copy(nrm_hbm.at[pl.ds(start, EB)], nrm_b, semn)

            def zc(j, c2):
                ix = pl.ds(j * 16, 16)
                cloc[ix] = z16i
                csrc[ix] = z16i
                cnrm[ix] = z16f
                return c2

            lax.fori_loop(0, CB // 16, zc, 0)
            dd.wait()
            ds_.wait()
            dn.wait()

            def vec(v, cur):
                ix = pl.ds(v * 16, 16)
                d = dst_b[ix]
                rel = d - lo
                m = (rel >= 0) & (rel < C)
                dsc = pl.ds(cur, 16)
                plsc.store_compressed(cloc.at[dsc], rel, mask=m)
                plsc.store_compressed(csrc.at[dsc], src_b[ix], mask=m)
                plsc.store_compressed(cnrm.at[dsc], nrm_b[ix], mask=m)
                return cur + plsc.all_reduce_population_count(m)[0]

            cur = lax.fori_loop(0, EB // 16, vec, 0)
            npair = (cur + 2 * G - 1) // (2 * G)
            lanes = lax.iota(jnp.int32, 16)

            def half(g, rows, sl):
                def scale(v2, c3):
                    w = cnrm[pl.ds(g * G + v2 * 16, 16)]
                    ri = lanes + v2 * 16
                    for f in range(D):
                        cf = jnp.full((16,), f, dtype=jnp.int32)
                        y = plsc.load_gather(rows, [ri, cf]) * w
                        plsc.store_scatter(rows, [ri, cf], y)
                    return c3

                lax.fori_loop(0, G // 16, scale, 0)
                for j in range(G // 16):
                    idx2d[sl, pl.ds(j * 16, 16)] = cloc[pl.ds(g * G + j * 16, 16)]
                pltpu.sync_copy(rows, chunk_sh.at[idx2d.at[sl]], add=True)

            def pair(gp, c2):
                g0 = 2 * gp
                g1 = g0 + 1
                d0 = pltpu.async_copy(h_hbm.at[csrc.at[pl.ds(g0 * G, G)]],
                                      rows_v, sem)
                d1 = pltpu.async_copy(h_hbm.at[csrc.at[pl.ds(g1 * G, G)]],
                                      rows_w, sem2)
                d0.wait()
                half(g0, rows_v, 0)
                d1.wait()
                half(g1, rows_w, 1)
                return c2

            lax.fori_loop(0, npair, pair, 0)
            return cc

        lax.fori_loop(0, nblk, block, 0)
        plsc.subcore_barrier()

        def wb(k, cc):
            r0 = sid * rpt + k * 64
            pltpu.sync_copy(chunk_sh.at[pl.ds(r0, 64)],
                            s_hbm.at[pl.ds(lo + r0, 64)])
            return cc

        lax.fori_loop(0, rpt // 64, wb, 0)
        return carry0

    lax.fori_loop(0, my_chunks, chunk_iter, 0)


def _make_spmm(D, C, SP):
    return pl.kernel(
        functools.partial(_spmm_body, D, C, SP),
        out_type=jax.ShapeDtypeStruct((SP, D), jnp.float32),
        mesh=plsc.VectorSubcoreMesh(
            core_axis_name="c", subcore_axis_name="s", num_cores=NC,
            num_subcores=NS),
        scratch_types=[
            pltpu.VMEM((EB,), jnp.int32),
            pltpu.VMEM((EB,), jnp.int32),
            pltpu.VMEM((EB,), jnp.float32),
            pltpu.VMEM((CB,), jnp.int32),
            pltpu.VMEM((CB,), jnp.int32),
            pltpu.VMEM((CB,), jnp.float32),
            pltpu.VMEM((2, G), jnp.int32),
            pltpu.VMEM((G, D), jnp.float32),
            pltpu.VMEM((G, D), jnp.float32),
            pltpu.VMEM_SHARED((C, D), jnp.float32),
            pltpu.SemaphoreType.DMA,
            pltpu.SemaphoreType.DMA,
            pltpu.SemaphoreType.DMA,
            pltpu.SemaphoreType.DMA,
            pltpu.SemaphoreType.DMA,
        ],
        compiler_params=pltpu.CompilerParams(needs_layout_passes=False),
    )


_spmm128 = _make_spmm(128, 10240, 102400)


def _head_kernel(code_ref, wl1_ref, bl1_ref, wl2_ref, bl2_ref, wl3_ref, bl3_ref,
                 z_ref):
    code = code_ref[...]
    z = jax.nn.relu(jnp.dot(code, wl1_ref[...]) + bl1_ref[...])
    z = jax.nn.relu(jnp.dot(z, wl2_ref[...]) + bl2_ref[...])
    z_ref[...] = jnp.dot(z, wl3_ref[...]) + bl3_ref[...]


def kernel(x, edge_index, edge_attr, batch, C2ER, W1, b1, W2, b2, W3, b3, W4,
           b4, Wl1, bl1, Wl2, bl2, Wl3, bl3):
    src = edge_index[0]
    dst = edge_index[1]

    degp = _deg_call(dst, edge_attr)
    deg = jnp.sum(jnp.reshape(degp, (NT, NP)), axis=0)[:N] + 1.0
    dinv = jnp.where(deg > 0, lax.rsqrt(jnp.maximum(deg, 1e-12)), 0.0)
    dinv2 = dinv * dinv
    norm = dinv[src] * edge_attr * dinv[dst]

    def gcn(h_in, W, b):
        h = h_in @ W
        d = h.shape[1]
        hpad = jnp.pad(h, ((0, NP - N), (0, 128 - d)))
        s = _spmm128(src, dst, norm, hpad)[:N, :d]
        return s + dinv2[:, None] * h + b

    h = jax.nn.relu(gcn(x, W1, b1))
    x0 = jnp.max(h, axis=0, keepdims=True)
    h = jax.nn.relu(gcn(h, W2, b2))
    h = jax.nn.relu(gcn(h, W3, b3))
    h = jax.nn.relu(gcn(h, W4, b4))
    x1 = jnp.max(h, axis=0, keepdims=True)
    code = jnp.concatenate([x0, x1, jnp.reshape(C2ER, (1, 4))], axis=1)
    logits = pl.pallas_call(
        _head_kernel,
        out_shape=jax.ShapeDtypeStruct((1, 10), jnp.float32),
    )(code, Wl1, jnp.reshape(bl1, (1, 128)), Wl2, jnp.reshape(bl2, (1, 128)),
      Wl3, jnp.reshape(bl3, (1, 10)))
    z = jax.nn.softmax(logits, axis=1)
    return (z, code)


# speedup vs baseline: 1.5717x; 1.5717x over previous
import functools

import jax
import jax.numpy as jnp
from jax import lax
from jax.experimental import pallas as pl
from jax.experimental.pallas import tpu as pltpu
from jax.experimental.pallas import tpu_sc as plsc

N = 100000
E = 1600000
NC = 2
NS = 16
NT = NC * NS
SL = 6272
NP = NS * SL
EB = 2560
NEB = E // EB
VPB = EB // 16


def _deg_body(dst_hbm, ew_hbm, out_hbm, deg_v, dst_b, ew_b):
    z16 = jnp.zeros((16,), jnp.float32)
    cid = lax.axis_index("c")
    sid = lax.axis_index("s")
    wid = sid * NC + cid

    def zero_deg(j, carry):
        deg_v[pl.ds(j * 16, 16)] = z16
        return carry

    lax.fori_loop(0, NP // 16, zero_deg, 0)

    nblk = jnp.where(wid < NEB % NT, NEB // NT + 1, NEB // NT)

    def block(b, carry):
        start = (wid + b * NT) * EB
        pltpu.sync_copy(dst_hbm.at[pl.ds(start, EB)], dst_b)
        pltpu.sync_copy(ew_hbm.at[pl.ds(start, EB)], ew_b)

        def vec(v, c2):
            d = dst_b[pl.ds(v * 16, 16)]
            w = ew_b[pl.ds(v * 16, 16)]
            plsc.addupdate_scatter(deg_v, [d], w)
            return c2

        return lax.fori_loop(0, VPB, vec, carry)

    lax.fori_loop(0, nblk, block, 0)
    pltpu.sync_copy(deg_v, out_hbm.at[pl.ds(wid * NP, NP)])


_deg_call = pl.kernel(
    _deg_body,
    out_type=jax.ShapeDtypeStruct((NT * NP,), jnp.float32),
    mesh=plsc.VectorSubcoreMesh(
        core_axis_name="c", subcore_axis_name="s", num_cores=NC,
        num_subcores=NS),
    scratch_types=[
        pltpu.VMEM((NP,), jnp.float32),
        pltpu.VMEM((EB,), jnp.int32),
        pltpu.VMEM((EB,), jnp.float32),
    ],
    compiler_params=pltpu.CompilerParams(needs_layout_passes=False),
)


G = 128
CB = EB + G


def _spmm_body(D, C, SP, src_hbm, dst_hbm, nrm_hbm, h_hbm, s_hbm,
               src_b, dst_b, nrm_b, csrc, cloc, cnrm, idx2d, rows_v,
               chunk_sh, sem, semd, sems, semn):
    z16f = jnp.zeros((16,), jnp.float32)
    z16i = jnp.zeros((16,), jnp.int32)
    cid = lax.axis_index("c")
    sid = lax.axis_index("s")
    nchunks = SP // C
    rpt = C // NS
    my_chunks = (nchunks - cid + NC - 1) // NC
    nblk = jnp.where(sid < NEB % NS, NEB // NS + 1, NEB // NS)

    def chunk_iter(ci, carry0):
        lo = (ci * NC + cid) * C

        def zrow(r, cc):
            for j in range(D // 16):
                rows_v[r, pl.ds(j * 16, 16)] = z16f
            return cc

        lax.fori_loop(0, G, zrow, 0)

        def zchunk(k, cc):
            pltpu.sync_copy(rows_v.at[pl.ds(0, 64)],
                            chunk_sh.at[pl.ds(sid * rpt + k * 64, 64)])
            return cc

        lax.fori_loop(0, rpt // 64, zchunk, 0)
        plsc.subcore_barrier()

        def block(b, cc):
            start = (sid + b * NS) * EB
            dd = pltpu.async_copy(dst_hbm.at[pl.ds(start, EB)], dst_b, semd)
            ds_ = pltpu.async_copy(src_hbm.at[pl.ds(start, EB)], src_b, sems)
            dn = pltpu.async_copy(nrm_hbm.at[pl.ds(start, EB)], nrm_b, semn)

            def zc(j, c2):
                ix = pl.ds(j * 16, 16)
                cloc[ix] = z16i
                csrc[ix] = z16i
                cnrm[ix] = z16f
                return c2

            lax.fori_loop(0, CB // 16, zc, 0)
            dd.wait()
            ds_.wait()
            dn.wait()

            def vec(v, cur):
                ix = pl.ds(v * 16, 16)
                d = dst_b[ix]
                rel = d - lo
                m = (rel >= 0) & (rel < C)
                dsc = pl.ds(cur, 16)
                plsc.store_compressed(cloc.at[dsc], rel, mask=m)
                plsc.store_compressed(csrc.at[dsc], src_b[ix], mask=m)
                plsc.store_compressed(cnrm.at[dsc], nrm_b[ix], mask=m)
                return cur + plsc.all_reduce_population_count(m)[0]

            cur = lax.fori_loop(0, EB // 16, vec, 0)
            ng = (cur + G - 1) // G
            lanes = lax.iota(jnp.int32, 16)

            def group(g, c2):
                pltpu.async_copy(h_hbm.at[csrc.at[pl.ds(g * G, G)]], rows_v,
                                 sem).wait()

                def scale(v2, c3):
                    w = cnrm[pl.ds(g * G + v2 * 16, 16)]
                    ri = lanes + v2 * 16
                    for f in range(D):
                        cf = jnp.full((16,), f, dtype=jnp.int32)
                        y = plsc.load_gather(rows_v, [ri, cf]) * w
                        plsc.store_scatter(rows_v, [ri, cf], y)
                    return c3

                lax.fori_loop(0, G // 16, scale, 0)
                for j in range(G // 16):
                    idx2d[0, pl.ds(j * 16, 16)] = cloc[pl.ds(g * G + j * 16, 16)]
                pltpu.sync_copy(rows_v, chunk_sh.at[idx2d.at[0]], add=True)
                return c2

            lax.fori_loop(0, ng, group, 0)
            return cc

        lax.fori_loop(0, nblk, block, 0)
        plsc.subcore_barrier()

        def wb(k, cc):
            r0 = sid * rpt + k * 64
            pltpu.sync_copy(chunk_sh.at[pl.ds(r0, 64)],
                            s_hbm.at[pl.ds(lo + r0, 64)])
            return cc

        lax.fori_loop(0, rpt // 64, wb, 0)
        return carry0

    lax.fori_loop(0, my_chunks, chunk_iter, 0)


def _make_spmm(D, C, SP):
    return pl.kernel(
        functools.partial(_spmm_body, D, C, SP),
        out_type=jax.ShapeDtypeStruct((SP, D), jnp.float32),
        mesh=plsc.VectorSubcoreMesh(
            core_axis_name="c", subcore_axis_name="s", num_cores=NC,
            num_subcores=NS),
        scratch_types=[
            pltpu.VMEM((EB,), jnp.int32),
            pltpu.VMEM((EB,), jnp.int32),
            pltpu.VMEM((EB,), jnp.float32),
            pltpu.VMEM((CB,), jnp.int32),
            pltpu.VMEM((CB,), jnp.int32),
            pltpu.VMEM((CB,), jnp.float32),
            pltpu.VMEM((2, G), jnp.int32),
            pltpu.VMEM((G, D), jnp.float32),
            pltpu.VMEM_SHARED((C, D), jnp.float32),
            pltpu.SemaphoreType.DMA,
            pltpu.SemaphoreType.DMA,
            pltpu.SemaphoreType.DMA,
            pltpu.SemaphoreType.DMA,
        ],
        compiler_params=pltpu.CompilerParams(needs_layout_passes=False),
    )


_spmm128 = _make_spmm(128, 11264, 101376)


def _head_kernel(code_ref, wl1_ref, bl1_ref, wl2_ref, bl2_ref, wl3_ref, bl3_ref,
                 z_ref):
    code = code_ref[...]
    z = jax.nn.relu(jnp.dot(code, wl1_ref[...]) + bl1_ref[...])
    z = jax.nn.relu(jnp.dot(z, wl2_ref[...]) + bl2_ref[...])
    z_ref[...] = jnp.dot(z, wl3_ref[...]) + bl3_ref[...]


def kernel(x, edge_index, edge_attr, batch, C2ER, W1, b1, W2, b2, W3, b3, W4,
           b4, Wl1, bl1, Wl2, bl2, Wl3, bl3):
    src = edge_index[0]
    dst = edge_index[1]

    degp = _deg_call(dst, edge_attr)
    deg = jnp.sum(jnp.reshape(degp, (NT, NP)), axis=0)[:N] + 1.0
    dinv = jnp.where(deg > 0, lax.rsqrt(jnp.maximum(deg, 1e-12)), 0.0)
    dinv2 = dinv * dinv
    norm = dinv[src] * edge_attr * dinv[dst]

    def gcn(h_in, W, b):
        h = h_in @ W
        d = h.shape[1]
        hpad = jnp.pad(h, ((0, NP - N), (0, 128 - d)))
        s = _spmm128(src, dst, norm, hpad)[:N, :d]
        return s + dinv2[:, None] * h + b

    h = jax.nn.relu(gcn(x, W1, b1))
    x0 = jnp.max(h, axis=0, keepdims=True)
    h = jax.nn.relu(gcn(h, W2, b2))
    h = jax.nn.relu(gcn(h, W3, b3))
    h = jax.nn.relu(gcn(h, W4, b4))
    x1 = jnp.max(h, axis=0, keepdims=True)
    code = jnp.concatenate([x0, x1, jnp.reshape(C2ER, (1, 4))], axis=1)
    logits = pl.pallas_call(
        _head_kernel,
        out_shape=jax.ShapeDtypeStruct((1, 10), jnp.float32),
    )(code, Wl1, jnp.reshape(bl1, (1, 128)), Wl2, jnp.reshape(bl2, (1, 128)),
      Wl3, jnp.reshape(bl3, (1, 10)))
    z = jax.nn.softmax(logits, axis=1)
    return (z, code)
